# baseline (speedup 1.0000x reference)
import functools

import jax
import jax.numpy as jnp
from jax import lax
from jax.experimental import pallas as pl
from jax.experimental.pallas import tpu as pltpu
from jax.experimental.pallas import tpu_sc as plsc

N = 10000
E = 320000
H = 128
G = 8
EPS = 1e-7

NCORE = 2
NSUB = 16
NP = 10240
HALF_N = 5008
ACC_E = 5008
B = 64
EDGES_PER_SUB = E // NSUB
NFULL = EDGES_PER_SUB // B
TAIL = EDGES_PER_SUB - NFULL * B
ZCH = B * 2 * H
ZPT = ACC_E * 2 * H // NSUB
WPT = HALF_N // NSUB * 2 * H


def _edge_body(src_hbm, dst_hbm, h_hbm, emb_hbm, out_hbm,
               src_v, dst_v, hsrc_v, emb_v, estage_v, eidx_v, accf, sem):
    c = lax.axis_index("c")
    s = lax.axis_index("s")
    lo = c * HALF_N

    def zs(i, _):
        estage_v[pl.ds(i * 16, 16)] = jnp.zeros((16,), jnp.float32)
        return 0
    lax.fori_loop(0, ZCH // 16, zs, 0)
    zb = s * ZPT
    for k in range(4):
        pltpu.sync_copy(estage_v, accf.at[pl.ds(zb + k * ZCH, ZCH)])
    pltpu.sync_copy(estage_v.at[pl.ds(0, ZPT - 4 * ZCH)],
                    accf.at[pl.ds(zb + 4 * ZCH, ZPT - 4 * ZCH)])
    plsc.subcore_barrier()

    iota = lax.broadcasted_iota(jnp.int32, (16,), 0)
    consts = [iota + j * 16 for j in range(8)]
    ebase = pl.multiple_of(s * EDGES_PER_SUB, 128)

    def do_batch(base, nload):
        pltpu.sync_copy(src_hbm.at[pl.ds(base, nload)], src_v.at[pl.ds(0, nload)])
        pltpu.sync_copy(dst_hbm.at[pl.ds(base, nload)], dst_v.at[pl.ds(0, nload)])
        pltpu.sync_copy(emb_hbm.at[pl.ds(base, nload)], emb_v.at[pl.ds(0, nload)])
        pltpu.async_copy(h_hbm.at[src_v], hsrc_v, sem).wait()
        for bk in range(nload // 16):
            d = dst_v[pl.ds(bk * 16, 16)]
            r = d - lo
            orm = jnp.where((r >= 0) & (r < HALF_N), 0, -1)
            rb = r * (2 * H)
            for k in range(16):
                ri = rb[k]
                rd = ri + H
                ork = orm[k]
                e = bk * 16 + k
                eb = e * (2 * H)
                for j in range(8):
                    hv = hsrc_v[e, pl.ds(j * 16, 16)]
                    ev = emb_v[e, pl.ds(j * 16, 16)]
                    m = jnp.maximum(hv + ev, 0.0) + EPS
                    x = jnp.exp(m)
                    estage_v[pl.ds(eb + j * 16, 16)] = m * x
                    estage_v[pl.ds(eb + H + j * 16, 16)] = x
                    eidx_v[pl.ds(eb + j * 16, 16)] = (consts[j] + ri) | ork
                    eidx_v[pl.ds(eb + H + j * 16, 16)] = (consts[j] + rd) | ork
        if nload < B:
            def zt(t, _):
                estage_v[pl.ds(nload * 2 * H + t * 16, 16)] = jnp.zeros(
                    (16,), jnp.float32)
                return 0
            lax.fori_loop(0, (B - nload) * 2 * H // 16, zt, 0)
        pltpu.async_copy(estage_v,
                         accf.at[plsc.Indices(eidx_v, ignored_value=-1)],
                         sem, add=True).wait()

    def bb(t, _):
        do_batch(pl.multiple_of(ebase + t * B, 8), B)
        return 0
    lax.fori_loop(0, NFULL, bb, 0)
    do_batch(pl.multiple_of(ebase + NFULL * B, 8), TAIL)

    plsc.subcore_barrier()
    pltpu.sync_copy(accf.at[pl.ds(s * WPT, WPT)],
                    out_hbm.at[pl.ds((lo + s * (HALF_N // NSUB)) * 2 * H, WPT)])


_edge_pass = functools.partial(
    pl.kernel,
    out_type=jax.ShapeDtypeStruct((NP * 2 * H,), jnp.float32),
    mesh=plsc.VectorSubcoreMesh(core_axis_name="c", subcore_axis_name="s"),
    scratch_types=[
        pltpu.VMEM((B,), jnp.int32),
        pltpu.VMEM((B,), jnp.int32),
        pltpu.VMEM((B, H), jnp.float32),
        pltpu.VMEM((B, H), jnp.float32),
        pltpu.VMEM((B * 2 * H,), jnp.float32),
        pltpu.VMEM((B * 2 * H,), jnp.int32),
        pltpu.VMEM_SHARED((ACC_E * 2 * H,), jnp.float32),
        pltpu.SemaphoreType.DMA,
    ],
)(_edge_body)


def _encx_body(x_ref, w_ref, b_ref, o_ref):
    o_ref[...] = jnp.dot(x_ref[...], w_ref[...],
                         preferred_element_type=jnp.float32) + b_ref[...]


def _node_body(res, ci_ref, hp_ref, nd_ref, w_ref, b_ref, g_ref, be_ref,
               hn_ref, cn_ref):
    num = nd_ref[:, :H]
    den = nd_ref[:, H:]
    aggr = num / (den + 1e-16)
    t = ci_ref[...] + aggr
    out = jnp.dot(t, w_ref[...], preferred_element_type=jnp.float32) + b_ref[...]
    hnew = out + hp_ref[...] if res else out
    hn_ref[...] = hnew
    mu = jnp.mean(hnew, axis=1, keepdims=True)
    d = hnew - mu
    var = jnp.mean(d * d, axis=1, keepdims=True)
    ln = d * lax.rsqrt(var + 1e-5) * g_ref[...] + be_ref[...]
    cn_ref[...] = jnp.maximum(ln, 0.0)


def _pool_body(hf_ref, b_ref, wh_ref, bh_ref, o_ref):
    ids = b_ref[...]
    g = lax.broadcasted_iota(jnp.int32, (G, 1), 0)
    oh = (ids == g).astype(jnp.float32)
    pooled = jnp.dot(oh, hf_ref[...], preferred_element_type=jnp.float32)
    counts = jnp.sum(oh, axis=1, keepdims=True)
    mp = pooled / jnp.maximum(counts, 1.0)
    r = jnp.sum(mp * wh_ref[...], axis=1, keepdims=True) + bh_ref[0, 0]
    o_ref[...] = jnp.broadcast_to(r, (G, H))


_EB = 8000
_NB = 2048


def _encode_x(xp, w, b):
    return pl.pallas_call(
        _encx_body,
        out_shape=jax.ShapeDtypeStruct((NP, H), jnp.float32),
    )(xp, w, b)


def _encode_e(a, w, b):
    return pl.pallas_call(
        _encx_body,
        grid=(E // _EB,),
        in_specs=[pl.BlockSpec((_EB, 7), lambda i: (i, 0)),
                  pl.BlockSpec((7, H), lambda i: (0, 0)),
                  pl.BlockSpec((1, H), lambda i: (0, 0))],
        out_specs=pl.BlockSpec((_EB, H), lambda i: (i, 0)),
        out_shape=jax.ShapeDtypeStruct((E, H), jnp.float32),
    )(a, w, b)


def _node_update(res, ci, hp, nd, w, b, gm, bt):
    return pl.pallas_call(
        functools.partial(_node_body, res),
        grid=(NP // _NB,),
        in_specs=[pl.BlockSpec((_NB, H), lambda i: (i, 0)),
                  pl.BlockSpec((_NB, H), lambda i: (i, 0)),
                  pl.BlockSpec((_NB, 2 * H), lambda i: (i, 0)),
                  pl.BlockSpec((H, H), lambda i: (0, 0)),
                  pl.BlockSpec((1, H), lambda i: (0, 0)),
                  pl.BlockSpec((1, H), lambda i: (0, 0)),
                  pl.BlockSpec((1, H), lambda i: (0, 0))],
        out_specs=[pl.BlockSpec((_NB, H), lambda i: (i, 0)),
                   pl.BlockSpec((_NB, H), lambda i: (i, 0))],
        out_shape=[jax.ShapeDtypeStruct((NP, H), jnp.float32),
                   jax.ShapeDtypeStruct((NP, H), jnp.float32)],
    )(ci, hp, nd, w, b, gm, bt)


def _pool(hf, batch2d, wh, bh):
    return pl.pallas_call(
        _pool_body,
        out_shape=jax.ShapeDtypeStruct((G, H), jnp.float32),
    )(hf, batch2d, wh, bh)


def kernel(x, edge_index, edge_attr, batch, W_node, b_node, W_edge, b_edge,
           W_mlp, b_mlp, gamma, beta, W_head, b_head):
    src = edge_index[0]
    dst = edge_index[1]
    xp = jnp.pad(x, ((0, NP - N), (0, 0)))
    h0 = _encode_x(xp, W_node, b_node.reshape(1, H))
    emb = _encode_e(edge_attr, W_edge, b_edge.reshape(1, H))
    conv_in = h0
    h = h0
    for l in range(3):
        nd = _edge_pass(src, dst, conv_in, emb).reshape(NP, 2 * H)
        h, conv_in = _node_update(l > 0, conv_in, h, nd,
                                  W_mlp[l], b_mlp[l].reshape(1, H),
                                  gamma[l].reshape(1, H), beta[l].reshape(1, H))
    batch_p = jnp.pad(batch, (0, NP - N), constant_values=G).reshape(1, NP)
    out = _pool(conv_in, batch_p, W_head.reshape(1, H), b_head.reshape(1, 1))
    return out[:, :1]


# speedup vs baseline: 1.9448x; 1.9448x over previous
import functools

import jax
import jax.numpy as jnp
from jax import lax
from jax.experimental import pallas as pl
from jax.experimental.pallas import tpu as pltpu
from jax.experimental.pallas import tpu_sc as plsc

N = 10000
E = 320000
H = 128
G = 8
EPS = 1e-7

NCORE = 2
NSUB = 16
NP = 10240
HALF_N = 5008
B = 32
EDGES_PER_SUB = E // NSUB
NBAT = EDGES_PER_SUB // B
ZPT = HALF_N * H // NSUB
ZCH = B * H


def _edge_body(src_hbm, dst_hbm, h_hbm, emb_hbm, outn_hbm, outd_hbm,
               src0, src1, dst0, dst1, hs0, hs1, em0, em1,
               stn0, stn1, std0, std1, ei0, ei1, accn, accd,
               sl0, sl1, sg0, sg1, ss0, ss1):
    c = lax.axis_index("c")
    s = lax.axis_index("s")
    lo = c * HALF_N
    SRC = [src0, src1]
    DST = [dst0, dst1]
    HS = [hs0, hs1]
    EM = [em0, em1]
    STN = [stn0, stn1]
    STD = [std0, std1]
    EI = [ei0, ei1]
    SL = [sl0, sl1]
    SG = [sg0, sg1]
    SS = [ss0, ss1]

    def zs(i, _):
        stn0[pl.ds(i * 16, 16)] = jnp.zeros((16,), jnp.float32)
        return 0
    lax.fori_loop(0, ZCH // 16, zs, 0)
    zb = s * ZPT
    for acc in (accn, accd):
        for k in range(9):
            pltpu.sync_copy(stn0, acc.at[pl.ds(zb + k * ZCH, ZCH)])
        pltpu.sync_copy(stn0.at[pl.ds(0, ZPT - 9 * ZCH)],
                        acc.at[pl.ds(zb + 9 * ZCH, ZPT - 9 * ZCH)])
    plsc.subcore_barrier()

    iota = lax.broadcasted_iota(jnp.int32, (16,), 0)
    consts = [iota + j * 16 for j in range(8)]
    ebase = pl.multiple_of(s * EDGES_PER_SUB, 8)
    maxb = E - B

    def issue_loads(t, p):
        base = pl.multiple_of(jnp.minimum(ebase + t * B, maxb), 8)
        pltpu.async_copy(src_hbm.at[pl.ds(base, B)], SRC[p], SL[p])
        pltpu.async_copy(dst_hbm.at[pl.ds(base, B)], DST[p], SL[p])
        pltpu.async_copy(emb_hbm.at[pl.ds(base, B)], EM[p], SL[p])

    def wait_loads(p):
        pltpu.make_async_copy(src_hbm.at[pl.ds(0, B)], SRC[p], SL[p]).wait()
        pltpu.make_async_copy(dst_hbm.at[pl.ds(0, B)], DST[p], SL[p]).wait()
        pltpu.make_async_copy(emb_hbm.at[pl.ds(0, B)], EM[p], SL[p]).wait()

    def issue_gather(p):
        pltpu.async_copy(h_hbm.at[SRC[p]], HS[p], SG[p])

    def wait_gather(p):
        pltpu.make_async_copy(h_hbm.at[SRC[p]], HS[p], SG[p]).wait()

    def issue_scatter(p):
        pltpu.async_copy(STN[p], accn.at[plsc.Indices(EI[p], ignored_value=-1)],
                         SS[p], add=True)
        pltpu.async_copy(STD[p], accd.at[plsc.Indices(EI[p], ignored_value=-1)],
                         SS[p], add=True)

    def wait_scatter(p):
        pltpu.make_async_copy(
            STN[p], accn.at[plsc.Indices(EI[p], ignored_value=-1)], SS[p]).wait()
        pltpu.make_async_copy(
            STD[p], accd.at[plsc.Indices(EI[p], ignored_value=-1)], SS[p]).wait()

    def compute(p):
        for bk in range(B // 16):
            d = DST[p][pl.ds(bk * 16, 16)]
            r = d - lo
            orm = jnp.where((r >= 0) & (r < HALF_N), 0, -1)
            rb = r * H
            for k in range(16):
                ri = rb[k]
                ork = orm[k]
                e = bk * 16 + k
                eb = e * H
                for j in range(8):
                    hv = HS[p][e, pl.ds(j * 16, 16)]
                    ev = EM[p][e, pl.ds(j * 16, 16)]
                    m = jnp.maximum(hv + ev, 0.0) + EPS
                    x = jnp.exp(m)
                    STN[p][pl.ds(eb + j * 16, 16)] = m * x
                    STD[p][pl.ds(eb + j * 16, 16)] = x
                    EI[p][pl.ds(eb + j * 16, 16)] = (consts[j] + ri) | ork

    def step(t, p, warm):
        q = 1 - p
        wait_loads(q)
        issue_gather(q)
        wait_gather(p)
        if warm:
            wait_scatter(p)
        compute(p)
        issue_scatter(p)
        issue_loads(t + 2, p)

    issue_loads(0, 0)
    wait_loads(0)
    issue_gather(0)
    issue_loads(1, 1)
    step(0, 0, False)
    step(1, 1, False)

    def pair(g, _):
        t = 2 + 2 * g
        step(t, 0, True)
        step(t + 1, 1, True)
        return 0
    lax.fori_loop(0, (NBAT - 3) // 2, pair, 0)

    wait_loads(1)
    wait_gather(0)
    wait_scatter(0)
    compute(0)
    issue_scatter(0)
    wait_scatter(1)
    wait_scatter(0)

    plsc.subcore_barrier()
    woff = s * ZPT
    pltpu.sync_copy(accn.at[pl.ds(woff, ZPT)],
                    outn_hbm.at[pl.ds(lo * H + woff, ZPT)])
    pltpu.sync_copy(accd.at[pl.ds(woff, ZPT)],
                    outd_hbm.at[pl.ds(lo * H + woff, ZPT)])


_edge_pass = functools.partial(
    pl.kernel,
    out_type=(jax.ShapeDtypeStruct((NP * H,), jnp.float32),
              jax.ShapeDtypeStruct((NP * H,), jnp.float32)),
    mesh=plsc.VectorSubcoreMesh(core_axis_name="c", subcore_axis_name="s"),
    scratch_types=(
        [pltpu.VMEM((B,), jnp.int32) for _ in range(4)]
        + [pltpu.VMEM((B, H), jnp.float32) for _ in range(4)]
        + [pltpu.VMEM((B * H,), jnp.float32) for _ in range(4)]
        + [pltpu.VMEM((B * H,), jnp.int32) for _ in range(2)]
        + [pltpu.VMEM_SHARED((HALF_N * H,), jnp.float32) for _ in range(2)]
        + [pltpu.SemaphoreType.DMA for _ in range(6)]
    ),
)(_edge_body)


def _encx_body(x_ref, w_ref, b_ref, o_ref):
    o_ref[...] = jnp.dot(x_ref[...], w_ref[...],
                         preferred_element_type=jnp.float32) + b_ref[...]


def _node_body(res, ci_ref, hp_ref, num_ref, den_ref, w_ref, b_ref, g_ref,
               be_ref, hn_ref, cn_ref):
    aggr = num_ref[...] / (den_ref[...] + 1e-16)
    t = ci_ref[...] + aggr
    out = jnp.dot(t, w_ref[...], preferred_element_type=jnp.float32) + b_ref[...]
    hnew = out + hp_ref[...] if res else out
    hn_ref[...] = hnew
    mu = jnp.mean(hnew, axis=1, keepdims=True)
    d = hnew - mu
    var = jnp.mean(d * d, axis=1, keepdims=True)
    ln = d * lax.rsqrt(var + 1e-5) * g_ref[...] + be_ref[...]
    cn_ref[...] = jnp.maximum(ln, 0.0)


def _pool_body(hf_ref, b_ref, wh_ref, bh_ref, o_ref):
    ids = b_ref[...]
    g = lax.broadcasted_iota(jnp.int32, (G, 1), 0)
    oh = (ids == g).astype(jnp.float32)
    pooled = jnp.dot(oh, hf_ref[...], preferred_element_type=jnp.float32)
    counts = jnp.sum(oh, axis=1, keepdims=True)
    mp = pooled / jnp.maximum(counts, 1.0)
    r = jnp.sum(mp * wh_ref[...], axis=1, keepdims=True) + bh_ref[0, 0]
    o_ref[...] = jnp.broadcast_to(r, (G, H))


_EB = 8000
_NB = 2048


def _encode_x(xp, w, b):
    return pl.pallas_call(
        _encx_body,
        out_shape=jax.ShapeDtypeStruct((NP, H), jnp.float32),
    )(xp, w, b)


def _encode_e(a, w, b):
    return pl.pallas_call(
        _encx_body,
        grid=(E // _EB,),
        in_specs=[pl.BlockSpec((_EB, 7), lambda i: (i, 0)),
                  pl.BlockSpec((7, H), lambda i: (0, 0)),
                  pl.BlockSpec((1, H), lambda i: (0, 0))],
        out_specs=pl.BlockSpec((_EB, H), lambda i: (i, 0)),
        out_shape=jax.ShapeDtypeStruct((E, H), jnp.float32),
    )(a, w, b)


def _node_update(res, ci, hp, num, den, w, b, gm, bt):
    return pl.pallas_call(
        functools.partial(_node_body, res),
        grid=(NP // _NB,),
        in_specs=[pl.BlockSpec((_NB, H), lambda i: (i, 0)),
                  pl.BlockSpec((_NB, H), lambda i: (i, 0)),
                  pl.BlockSpec((_NB, H), lambda i: (i, 0)),
                  pl.BlockSpec((_NB, H), lambda i: (i, 0)),
                  pl.BlockSpec((H, H), lambda i: (0, 0)),
                  pl.BlockSpec((1, H), lambda i: (0, 0)),
                  pl.BlockSpec((1, H), lambda i: (0, 0)),
                  pl.BlockSpec((1, H), lambda i: (0, 0))],
        out_specs=[pl.BlockSpec((_NB, H), lambda i: (i, 0)),
                   pl.BlockSpec((_NB, H), lambda i: (i, 0))],
        out_shape=[jax.ShapeDtypeStruct((NP, H), jnp.float32),
                   jax.ShapeDtypeStruct((NP, H), jnp.float32)],
    )(ci, hp, num, den, w, b, gm, bt)


def _pool(hf, batch2d, wh, bh):
    return pl.pallas_call(
        _pool_body,
        out_shape=jax.ShapeDtypeStruct((G, H), jnp.float32),
    )(hf, batch2d, wh, bh)


def kernel(x, edge_index, edge_attr, batch, W_node, b_node, W_edge, b_edge,
           W_mlp, b_mlp, gamma, beta, W_head, b_head):
    src = edge_index[0]
    dst = edge_index[1]
    xp = jnp.pad(x, ((0, NP - N), (0, 0)))
    h0 = _encode_x(xp, W_node, b_node.reshape(1, H))
    emb = _encode_e(edge_attr, W_edge, b_edge.reshape(1, H))
    conv_in = h0
    h = h0
    for l in range(3):
        numf, denf = _edge_pass(src, dst, conv_in, emb)
        h, conv_in = _node_update(l > 0, conv_in, h,
                                  numf.reshape(NP, H), denf.reshape(NP, H),
                                  W_mlp[l], b_mlp[l].reshape(1, H),
                                  gamma[l].reshape(1, H), beta[l].reshape(1, H))
    batch_p = jnp.pad(batch, (0, NP - N), constant_values=G).reshape(1, NP)
    out = _pool(conv_in, batch_p, W_head.reshape(1, H), b_head.reshape(1, 1))
    return out[:, :1]


# speedup vs baseline: 1.9509x; 1.0031x over previous
import functools

import jax
import jax.numpy as jnp
from jax import lax
from jax.experimental import pallas as pl
from jax.experimental.pallas import tpu as pltpu
from jax.experimental.pallas import tpu_sc as plsc

N = 10000
E = 320000
H = 128
G = 8
EPS = 1e-7
LOG2E = 1.4426950408889634
LN2 = 0.6931471805599453
C3 = 1.0 / 6.0
C4 = 1.0 / 24.0
C5 = 1.0 / 120.0
P16 = float(2.0 ** 16)
P32 = float(2.0 ** 32)
P48 = float(2.0 ** 48)
P64 = float(2.0 ** 64)
P80 = float(2.0 ** 80)
P96 = float(2.0 ** 96)
P112 = float(2.0 ** 112)

NCORE = 2
NSUB = 16
NP = 10240
HALF_N = 5008
B = 32
EDGES_PER_SUB = E // NSUB
NBAT = EDGES_PER_SUB // B
ZPT = HALF_N * H // NSUB
ZCH = B * H


def _edge_body(src_hbm, dst_hbm, h_hbm, emb_hbm, outn_hbm, outd_hbm,
               src0, src1, dst0, dst1, hs0, hs1, em0, em1,
               stn0, stn1, std0, std1, ei0, ei1, accn, accd,
               sl0, sl1, sg0, sg1, ss0, ss1):
    c = lax.axis_index("c")
    s = lax.axis_index("s")
    lo = c * HALF_N
    SRC = [src0, src1]
    DST = [dst0, dst1]
    HS = [hs0, hs1]
    EM = [em0, em1]
    STN = [stn0, stn1]
    STD = [std0, std1]
    EI = [ei0, ei1]
    SL = [sl0, sl1]
    SG = [sg0, sg1]
    SS = [ss0, ss1]

    def zs(i, _):
        stn0[pl.ds(i * 16, 16)] = jnp.zeros((16,), jnp.float32)
        return 0
    lax.fori_loop(0, ZCH // 16, zs, 0)
    zb = s * ZPT
    for acc in (accn, accd):
        for k in range(9):
            pltpu.sync_copy(stn0, acc.at[pl.ds(zb + k * ZCH, ZCH)])
        pltpu.sync_copy(stn0.at[pl.ds(0, ZPT - 9 * ZCH)],
                        acc.at[pl.ds(zb + 9 * ZCH, ZPT - 9 * ZCH)])
    plsc.subcore_barrier()

    iota = lax.broadcasted_iota(jnp.int32, (16,), 0)
    ebase = pl.multiple_of(s * EDGES_PER_SUB, 8)
    maxb = E - B

    def issue_loads(t, p):
        base = pl.multiple_of(jnp.minimum(ebase + t * B, maxb), 8)
        pltpu.async_copy(src_hbm.at[pl.ds(base, B)], SRC[p], SL[p])
        pltpu.async_copy(dst_hbm.at[pl.ds(base, B)], DST[p], SL[p])
        pltpu.async_copy(emb_hbm.at[pl.ds(base, B)], EM[p], SL[p])

    def wait_loads(p):
        pltpu.make_async_copy(src_hbm.at[pl.ds(0, B)], SRC[p], SL[p]).wait()
        pltpu.make_async_copy(dst_hbm.at[pl.ds(0, B)], DST[p], SL[p]).wait()
        pltpu.make_async_copy(emb_hbm.at[pl.ds(0, B)], EM[p], SL[p]).wait()

    def issue_gather(p):
        pltpu.async_copy(h_hbm.at[SRC[p]], HS[p], SG[p])

    def wait_gather(p):
        pltpu.make_async_copy(h_hbm.at[SRC[p]], HS[p], SG[p]).wait()

    def issue_scatter(p):
        pltpu.async_copy(STN[p], accn.at[plsc.Indices(EI[p], ignored_value=-1)],
                         SS[p], add=True)
        pltpu.async_copy(STD[p], accd.at[plsc.Indices(EI[p], ignored_value=-1)],
                         SS[p], add=True)

    def wait_scatter(p):
        pltpu.make_async_copy(
            STN[p], accn.at[plsc.Indices(EI[p], ignored_value=-1)], SS[p]).wait()
        pltpu.make_async_copy(
            STD[p], accd.at[plsc.Indices(EI[p], ignored_value=-1)], SS[p]).wait()

    def compute(p):
        def c1(t, _):
            for u in range(2):
                tt = t * 2 + u
                e = tt >> 3
                o16 = pl.multiple_of((tt & 7) * 16, 16)
                hv = HS[p][e, pl.ds(o16, 16)]
                ev = EM[p][e, pl.ds(o16, 16)]
                m = jnp.maximum(hv + ev, 0.0) + EPS
                kk = (m * LOG2E + 0.5).astype(jnp.int32)
                rr = m - kk.astype(jnp.float32) * LN2
                pp = 1.0 + rr * (1.0 + rr * (0.5 + rr * (C3 + rr * (
                    C4 + rr * C5))))
                x = pp * lax.bitcast_convert_type((kk + 127) << 23, jnp.float32)
                off = pl.multiple_of(tt * 16, 16)
                STN[p][pl.ds(off, 16)] = m * x
                STD[p][pl.ds(off, 16)] = x
            return 0
        lax.fori_loop(0, B * H // 32, c1, 0)
        for bk in range(B // 16):
            d = DST[p][pl.ds(bk * 16, 16)]
            r = d - lo
            orm = jnp.where((r >= 0) & (r < HALF_N), 0, -1)
            rb = r * H
            for k in range(16):
                ri = rb[k]
                ork = orm[k]
                eb = (bk * 16 + k) * H
                for j in range(8):
                    EI[p][pl.ds(eb + j * 16, 16)] = (iota + j * 16 + ri) | ork

    def step(t, p, warm):
        q = 1 - p
        wait_loads(q)
        issue_gather(q)
        wait_gather(p)
        if warm:
            wait_scatter(p)
        compute(p)
        issue_scatter(p)
        issue_loads(t + 2, p)

    issue_loads(0, 0)
    wait_loads(0)
    issue_gather(0)
    issue_loads(1, 1)
    step(0, 0, False)
    step(1, 1, False)

    def pair(g, _):
        t = 2 + 2 * g
        step(t, 0, True)
        step(t + 1, 1, True)
        return 0
    lax.fori_loop(0, (NBAT - 3) // 2, pair, 0)

    wait_loads(1)
    wait_gather(0)
    wait_scatter(0)
    compute(0)
    issue_scatter(0)
    wait_scatter(1)
    wait_scatter(0)

    plsc.subcore_barrier()
    woff = s * ZPT
    pltpu.sync_copy(accn.at[pl.ds(woff, ZPT)],
                    outn_hbm.at[pl.ds(lo * H + woff, ZPT)])
    pltpu.sync_copy(accd.at[pl.ds(woff, ZPT)],
                    outd_hbm.at[pl.ds(lo * H + woff, ZPT)])


_edge_pass = functools.partial(
    pl.kernel,
    out_type=(jax.ShapeDtypeStruct((NP * H,), jnp.float32),
              jax.ShapeDtypeStruct((NP * H,), jnp.float32)),
    mesh=plsc.VectorSubcoreMesh(core_axis_name="c", subcore_axis_name="s"),
    scratch_types=(
        [pltpu.VMEM((B,), jnp.int32) for _ in range(4)]
        + [pltpu.VMEM((B, H), jnp.float32) for _ in range(4)]
        + [pltpu.VMEM((B * H,), jnp.float32) for _ in range(4)]
        + [pltpu.VMEM((B * H,), jnp.int32) for _ in range(2)]
        + [pltpu.VMEM_SHARED((HALF_N * H,), jnp.float32) for _ in range(2)]
        + [pltpu.SemaphoreType.DMA for _ in range(6)]
    ),
)(_edge_body)


def _encx_body(x_ref, w_ref, b_ref, o_ref):
    o_ref[...] = jnp.dot(x_ref[...], w_ref[...],
                         preferred_element_type=jnp.float32) + b_ref[...]


def _node_body(res, ci_ref, hp_ref, num_ref, den_ref, w_ref, b_ref, g_ref,
               be_ref, hn_ref, cn_ref):
    aggr = num_ref[...] / (den_ref[...] + 1e-16)
    t = ci_ref[...] + aggr
    out = jnp.dot(t, w_ref[...], preferred_element_type=jnp.float32) + b_ref[...]
    hnew = out + hp_ref[...] if res else out
    hn_ref[...] = hnew
    mu = jnp.mean(hnew, axis=1, keepdims=True)
    d = hnew - mu
    var = jnp.mean(d * d, axis=1, keepdims=True)
    ln = d * lax.rsqrt(var + 1e-5) * g_ref[...] + be_ref[...]
    cn_ref[...] = jnp.maximum(ln, 0.0)


def _pool_body(hf_ref, b_ref, wh_ref, bh_ref, o_ref):
    ids = b_ref[...]
    g = lax.broadcasted_iota(jnp.int32, (G, 1), 0)
    oh = (ids == g).astype(jnp.float32)
    pooled = jnp.dot(oh, hf_ref[...], preferred_element_type=jnp.float32)
    counts = jnp.sum(oh, axis=1, keepdims=True)
    mp = pooled / jnp.maximum(counts, 1.0)
    r = jnp.sum(mp * wh_ref[...], axis=1, keepdims=True) + bh_ref[0, 0]
    o_ref[...] = jnp.broadcast_to(r, (G, H))


_EB = 8000
_NB = 2048


def _encode_x(xp, w, b):
    return pl.pallas_call(
        _encx_body,
        out_shape=jax.ShapeDtypeStruct((NP, H), jnp.float32),
    )(xp, w, b)


def _encode_e(a, w, b):
    return pl.pallas_call(
        _encx_body,
        grid=(E // _EB,),
        in_specs=[pl.BlockSpec((_EB, 7), lambda i: (i, 0)),
                  pl.BlockSpec((7, H), lambda i: (0, 0)),
                  pl.BlockSpec((1, H), lambda i: (0, 0))],
        out_specs=pl.BlockSpec((_EB, H), lambda i: (i, 0)),
        out_shape=jax.ShapeDtypeStruct((E, H), jnp.float32),
    )(a, w, b)


def _node_update(res, ci, hp, num, den, w, b, gm, bt):
    return pl.pallas_call(
        functools.partial(_node_body, res),
        grid=(NP // _NB,),
        in_specs=[pl.BlockSpec((_NB, H), lambda i: (i, 0)),
                  pl.BlockSpec((_NB, H), lambda i: (i, 0)),
                  pl.BlockSpec((_NB, H), lambda i: (i, 0)),
                  pl.BlockSpec((_NB, H), lambda i: (i, 0)),
                  pl.BlockSpec((H, H), lambda i: (0, 0)),
                  pl.BlockSpec((1, H), lambda i: (0, 0)),
                  pl.BlockSpec((1, H), lambda i: (0, 0)),
                  pl.BlockSpec((1, H), lambda i: (0, 0))],
        out_specs=[pl.BlockSpec((_NB, H), lambda i: (i, 0)),
                   pl.BlockSpec((_NB, H), lambda i: (i, 0))],
        out_shape=[jax.ShapeDtypeStruct((NP, H), jnp.float32),
                   jax.ShapeDtypeStruct((NP, H), jnp.float32)],
    )(ci, hp, num, den, w, b, gm, bt)


def _pool(hf, batch2d, wh, bh):
    return pl.pallas_call(
        _pool_body,
        out_shape=jax.ShapeDtypeStruct((G, H), jnp.float32),
    )(hf, batch2d, wh, bh)


def kernel(x, edge_index, edge_attr, batch, W_node, b_node, W_edge, b_edge,
           W_mlp, b_mlp, gamma, beta, W_head, b_head):
    src = edge_index[0]
    dst = edge_index[1]
    xp = jnp.pad(x, ((0, NP - N), (0, 0)))
    h0 = _encode_x(xp, W_node, b_node.reshape(1, H))
    emb = _encode_e(edge_attr, W_edge, b_edge.reshape(1, H))
    conv_in = h0
    h = h0
    for l in range(3):
        numf, denf = _edge_pass(src, dst, conv_in, emb)
        h, conv_in = _node_update(l > 0, conv_in, h,
                                  numf.reshape(NP, H), denf.reshape(NP, H),
                                  W_mlp[l], b_mlp[l].reshape(1, H),
                                  gamma[l].reshape(1, H), beta[l].reshape(1, H))
    batch_p = jnp.pad(batch, (0, NP - N), constant_values=G).reshape(1, NP)
    out = _pool(conv_in, batch_p, W_head.reshape(1, H), b_head.reshape(1, 1))
    return out[:, :1]
